# Initial kernel scaffold; baseline (speedup 1.0000x reference)
import jax
import jax.numpy as jnp
from jax.experimental import pallas as pl


def kernel(x, edge_index, batch, W1, b1, W2, b2, W3, b3, Wl, bl):
    raise NotImplementedError("write your pallas kernel here")


# speedup vs baseline: 20.1344x; 20.1344x over previous
import functools

import jax
import jax.numpy as jnp
from jax import lax
from jax.experimental import pallas as pl
from jax.experimental.pallas import tpu as pltpu
from jax.experimental.pallas import tpu_sc as plsc

_NC = 2
_NS = 16
_NW = _NC * _NS

_K = 80


def _sc_degree(n_nodes_pad, n_chunks):
    mesh = plsc.VectorSubcoreMesh(
        core_axis_name="c", subcore_axis_name="s",
        num_cores=_NC, num_subcores=_NS)
    rows_per_tile = n_nodes_pad // _NS

    @functools.partial(
        pl.kernel,
        out_type=jax.ShapeDtypeStruct((_NC * n_nodes_pad,), jnp.float32),
        mesh=mesh,
        scratch_types=[
            pltpu.VMEM_SHARED((n_nodes_pad,), jnp.float32),
            pltpu.VMEM((n_chunks, _K), jnp.int32),
            pltpu.VMEM((_K,), jnp.float32),
            pltpu.SemaphoreType.DMA,
            pltpu.SemaphoreType.DMA,
        ],
    )
    def deg_kernel(dst_hbm, ones_hbm, out_hbm, acc_sh, idx_v, ones_v,
                   ssem0, ssem1):
        cid = lax.axis_index("c")
        sid = lax.axis_index("s")
        wid = cid * _NS + sid
        r0 = pl.multiple_of(sid * rows_per_tile, 128)
        pltpu.sync_copy(ones_hbm.at[pl.ds(r0, rows_per_tile)],
                        acc_sh.at[pl.ds(r0, rows_per_tile)])
        pltpu.sync_copy(dst_hbm.at[wid], idx_v)
        pltpu.sync_copy(ones_hbm.at[pl.ds(0, _K)], ones_v)
        plsc.subcore_barrier()

        ssems = (ssem0, ssem1)

        def swait(j, s):
            pltpu.make_async_copy(ones_v, acc_sh.at[idx_v.at[j]],
                                  ssems[s]).wait()

        pltpu.async_copy(ones_v, acc_sh.at[idx_v.at[0]], ssems[0], add=True)

        def body(i, carry):
            for p in range(2):
                j = 2 * i + 1 + p
                swait(j - 1, p)
                pltpu.async_copy(ones_v, acc_sh.at[idx_v.at[j]],
                                 ssems[(1 + p) % 2], add=True)
            return carry

        n2 = (n_chunks - 1) // 2
        lax.fori_loop(0, n2, body, 0)
        for j in range(2 * n2 + 1, n_chunks):
            swait(j - 1, (j - 1) % 2)
            pltpu.async_copy(ones_v, acc_sh.at[idx_v.at[j]],
                             ssems[j % 2], add=True)
        swait(n_chunks - 1, (n_chunks - 1) % 2)
        plsc.subcore_barrier()
        o0 = pl.multiple_of(cid * n_nodes_pad + r0, 128)
        pltpu.sync_copy(acc_sh.at[pl.ds(r0, rows_per_tile)],
                        out_hbm.at[pl.ds(o0, rows_per_tile)])

    return deg_kernel


def _sc_edge_pass(n_nodes, d_feat, n_chunks):
    mesh = plsc.VectorSubcoreMesh(
        core_axis_name="c", subcore_axis_name="s",
        num_cores=_NC, num_subcores=_NS)
    dh = d_feat // 2
    rows_main = (n_nodes // (8 * _NS)) * 8
    rows_last = n_nodes - rows_main * (_NS - 1)
    assert rows_last % 8 == 0

    @functools.partial(
        pl.kernel,
        out_type=(jax.ShapeDtypeStruct((_NC, n_nodes, dh), jnp.float32),
                  jax.ShapeDtypeStruct((_NC, n_nodes, dh), jnp.float32)),
        mesh=mesh,
        compiler_params=pltpu.CompilerParams(use_tc_tiling_on_sc=False),
        scratch_types=[
            [pltpu.VMEM_SHARED((n_nodes + 8, dh), jnp.float32)] * 2,
            [pltpu.VMEM((2, _K), jnp.int32)] * 4,
            [[pltpu.VMEM((_K, dh), jnp.float32)] * 4] * 2,
            [pltpu.SemaphoreType.DMA] * 4,
            [[pltpu.SemaphoreType.DMA] * 4] * 2,
            [[pltpu.SemaphoreType.DMA] * 4] * 2,
        ],
    )
    def edge_kernel(gl_hbm, gr_hbm, zeros_hbm, ei_hbm, outl_hbm, outr_hbm,
                    accs, idxs, bufs2, isems, gsems2, ssems2):
        cid = lax.axis_index("c")
        sid = lax.axis_index("s")
        wid = cid * _NS + sid
        r0 = pl.multiple_of(sid * rows_main, 8)
        g2 = (gl_hbm, gr_hbm)
        out2 = (outl_hbm, outr_hbm)

        for h in range(2):
            for c, init_hbm in ((0, g2[h]), (1, zeros_hbm)):
                @pl.when(jnp.logical_and(cid == c, sid < _NS - 1))
                def _():
                    pltpu.sync_copy(init_hbm.at[pl.ds(r0, rows_main)],
                                    accs[h].at[pl.ds(r0, rows_main)])

                @pl.when(jnp.logical_and(cid == c, sid == _NS - 1))
                def _():
                    pltpu.sync_copy(
                        init_hbm.at[pl.ds(rows_main * (_NS - 1), rows_last)],
                        accs[h].at[pl.ds(rows_main * (_NS - 1), rows_last)])

        plsc.subcore_barrier()

        def wait_idx(j, s):
            pltpu.make_async_copy(ei_hbm.at[wid, j], idxs[s], isems[s]).wait()

        def issue_gathers(s):
            for h in range(2):
                pltpu.async_copy(g2[h].at[idxs[s].at[0]], bufs2[h][s],
                                 gsems2[h][s])

        def wait_gathers(s):
            for h in range(2):
                pltpu.make_async_copy(g2[h].at[idxs[s].at[0]], bufs2[h][s],
                                      gsems2[h][s]).wait()

        def issue_scatters(s):
            for h in range(2):
                pltpu.async_copy(bufs2[h][s], accs[h].at[idxs[s].at[1]],
                                 ssems2[h][s], add=True)

        def wait_scatters(s):
            for h in range(2):
                pltpu.make_async_copy(bufs2[h][s], accs[h].at[idxs[s].at[1]],
                                      ssems2[h][s]).wait()

        pltpu.async_copy(ei_hbm.at[wid, 0], idxs[0], isems[0])
        if n_chunks > 1:
            pltpu.async_copy(ei_hbm.at[wid, 1], idxs[1], isems[1])
        wait_idx(0, 0)
        issue_gathers(0)

        def step(j, p, static):
            if static:
                if j + 1 < n_chunks:
                    wait_idx(j + 1, (p + 1) % 4)
                    issue_gathers((p + 1) % 4)
                wait_gathers(p)
                if j >= 1:
                    wait_scatters((p + 3) % 4)
                issue_scatters(p)
                if j + 2 < n_chunks:
                    pltpu.async_copy(ei_hbm.at[wid, j + 2], idxs[(p + 2) % 4],
                                     isems[(p + 2) % 4])
            else:
                wait_idx(j + 1, (p + 1) % 4)
                issue_gathers((p + 1) % 4)
                wait_gathers(p)

                @pl.when(j >= 1)
                def _():
                    wait_scatters((p + 3) % 4)

                issue_scatters(p)

                @pl.when(j + 2 < n_chunks)
                def _():
                    pltpu.async_copy(ei_hbm.at[wid, j + 2], idxs[(p + 2) % 4],
                                     isems[(p + 2) % 4])

        n4 = (n_chunks - 1) // 4

        def body4(i, carry):
            for p in range(4):
                step(4 * i + p, p, False)
            return carry

        lax.fori_loop(0, n4, body4, 0)
        for j in range(4 * n4, n_chunks):
            step(j, j % 4, True)
        wait_scatters((n_chunks - 1) % 4)
        plsc.subcore_barrier()

        for h in range(2):
            @pl.when(sid < _NS - 1)
            def _():
                pltpu.sync_copy(accs[h].at[pl.ds(r0, rows_main)],
                                out2[h].at[cid, pl.ds(r0, rows_main)])

            @pl.when(sid == _NS - 1)
            def _():
                pltpu.sync_copy(
                    accs[h].at[pl.ds(rows_main * (_NS - 1), rows_last)],
                    out2[h].at[cid, pl.ds(rows_main * (_NS - 1), rows_last)])

    return edge_kernel


def _tc_first(x, w1, d0, d1, block_rows):
    n, d = x.shape
    dh = d // 2
    grid = n // block_rows

    def kern(x_ref, w_ref, d0_ref, d1_ref, gl_ref, gr_ref, dinv_ref):
        deg = d0_ref[...] + d1_ref[...] - 1.0
        dinv = lax.rsqrt(deg)
        g = jnp.dot(x_ref[...], w_ref[...],
                    preferred_element_type=jnp.float32) * dinv
        gl_ref[...] = g[:, :dh]
        gr_ref[...] = g[:, dh:]
        dinv_ref[...] = dinv

    return pl.pallas_call(
        kern,
        grid=(grid,),
        in_specs=[
            pl.BlockSpec((block_rows, d), lambda i: (i, 0)),
            pl.BlockSpec((d, d), lambda i: (0, 0)),
            pl.BlockSpec((block_rows, 1), lambda i: (i, 0)),
            pl.BlockSpec((block_rows, 1), lambda i: (i, 0)),
        ],
        out_specs=[
            pl.BlockSpec((block_rows, dh), lambda i: (i, 0)),
            pl.BlockSpec((block_rows, dh), lambda i: (i, 0)),
            pl.BlockSpec((block_rows, 1), lambda i: (i, 0)),
        ],
        out_shape=[
            jax.ShapeDtypeStruct((n, dh), jnp.float32),
            jax.ShapeDtypeStruct((n, dh), jnp.float32),
            jax.ShapeDtypeStruct((n, 1), jnp.float32),
        ],
    )(x, w1, d0, d1)


def _tc_mid(al, ar, dinv, b, w, block_rows):
    n, dh = al[0].shape
    d = 2 * dh
    grid = n // block_rows

    def kern(al0_ref, al1_ref, ar0_ref, ar1_ref, dinv_ref, b_ref, w_ref,
             gl_ref, gr_ref):
        dinv = dinv_ref[...]
        acc = jnp.concatenate(
            [al0_ref[...] + al1_ref[...], ar0_ref[...] + ar1_ref[...]],
            axis=1)
        h = jnp.maximum(acc * dinv + b_ref[...], 0.0)
        g = jnp.dot(h, w_ref[...],
                    preferred_element_type=jnp.float32) * dinv
        gl_ref[...] = g[:, :dh]
        gr_ref[...] = g[:, dh:]

    half = pl.BlockSpec((block_rows, dh), lambda i: (i, 0))
    return pl.pallas_call(
        kern,
        grid=(grid,),
        in_specs=[
            half, half, half, half,
            pl.BlockSpec((block_rows, 1), lambda i: (i, 0)),
            pl.BlockSpec((1, d), lambda i: (0, 0)),
            pl.BlockSpec((d, d), lambda i: (0, 0)),
        ],
        out_specs=[half, half],
        out_shape=[
            jax.ShapeDtypeStruct((n, dh), jnp.float32),
            jax.ShapeDtypeStruct((n, dh), jnp.float32),
        ],
    )(al[0], al[1], ar[0], ar[1], dinv, b, w)


def _tc_final(al, ar, dinv, b, wl, bl, batch3d, n_graphs, block_rows):
    n, dh = al[0].shape
    d = 2 * dh
    grid = n // block_rows

    def kern(al0_ref, al1_ref, ar0_ref, ar1_ref, dinv_ref, b_ref, wl_ref,
             bl_ref, batch_ref, out_ref):
        i = pl.program_id(0)
        acc = jnp.concatenate(
            [al0_ref[...] + al1_ref[...], ar0_ref[...] + ar1_ref[...]],
            axis=1)
        h = jnp.maximum(acc * dinv_ref[...] + b_ref[...], 0.0)
        s = jnp.dot(h, wl_ref[...], preferred_element_type=jnp.float32)
        bidx = batch_ref[0, 0, :]
        gids = lax.broadcasted_iota(jnp.int32, (block_rows, n_graphs), 1)
        onehot = (bidx[:, None] == gids).astype(jnp.float32)
        partial = lax.dot_general(
            onehot, s, (((0,), (0,)), ((), ())),
            preferred_element_type=jnp.float32)

        @pl.when(i == 0)
        def _():
            out_ref[...] = jnp.broadcast_to(bl_ref[...], (n_graphs, 1))

        out_ref[...] += partial

    half = pl.BlockSpec((block_rows, dh), lambda i: (i, 0))
    return pl.pallas_call(
        kern,
        grid=(grid,),
        in_specs=[
            half, half, half, half,
            pl.BlockSpec((block_rows, 1), lambda i: (i, 0)),
            pl.BlockSpec((1, d), lambda i: (0, 0)),
            pl.BlockSpec((d, 1), lambda i: (0, 0)),
            pl.BlockSpec((1, 1), lambda i: (0, 0)),
            pl.BlockSpec((1, 1, block_rows), lambda i: (i, 0, 0)),
        ],
        out_specs=pl.BlockSpec((n_graphs, 1), lambda i: (0, 0)),
        out_shape=jax.ShapeDtypeStruct((n_graphs, 1), jnp.float32),
    )(al[0], al[1], ar[0], ar[1], dinv, b, wl, bl, batch3d)


def kernel(x, edge_index, batch, W1, b1, W2, b2, W3, b3, Wl, bl):
    n, d = x.shape
    n_edges = edge_index.shape[1]
    n_graphs = 64

    epw = n_edges // _NW
    assert epw * _NW == n_edges
    n_chunks = -(-epw // _K)
    pad = n_chunks * _K - epw

    n_pad = ((n + _NS * 128 - 1) // (_NS * 128)) * (_NS * 128)

    src2 = edge_index[0].reshape(_NW, epw)
    dst2 = edge_index[1].reshape(_NW, epw)
    if pad:
        src2 = jnp.pad(src2, ((0, 0), (0, pad)))
        dst2 = jnp.pad(dst2, ((0, 0), (0, pad)), constant_values=n)
    ei = jnp.concatenate([src2.reshape(_NW, n_chunks, 1, _K),
                         dst2.reshape(_NW, n_chunks, 1, _K)], axis=2)
    dst3 = dst2.reshape(_NW, n_chunks, _K)
    ones = jnp.ones((n_pad,), jnp.float32)

    degs = _sc_degree(n_pad, n_chunks)(dst3, ones)
    d0 = degs[:n].reshape(n, 1)
    d1 = degs[n_pad:n_pad + n].reshape(n, 1)

    block_rows = 2000
    b1r = b1.reshape(1, d)
    b2r = b2.reshape(1, d)
    b3r = b3.reshape(1, d)
    blr = bl.reshape(1, 1)
    batch3d = batch.reshape(n // block_rows, 1, block_rows)

    edge_pass = _sc_edge_pass(n, d, n_chunks)
    zeros = jnp.zeros((n, d // 2), jnp.float32)

    gl, gr, dinv = _tc_first(x, W1, d0, d1, block_rows)
    al, ar = edge_pass(gl, gr, zeros, ei)
    gl, gr = _tc_mid(al, ar, dinv, b1r, W2, block_rows)
    al, ar = edge_pass(gl, gr, zeros, ei)
    gl, gr = _tc_mid(al, ar, dinv, b2r, W3, block_rows)
    al, ar = edge_pass(gl, gr, zeros, ei)
    out = _tc_final(al, ar, dinv, b3r, Wl, blr, batch3d,
                    n_graphs, block_rows)
    return out


# speedup vs baseline: 24.3991x; 1.2118x over previous
import functools

import jax
import jax.numpy as jnp
from jax import lax
from jax.experimental import pallas as pl
from jax.experimental.pallas import tpu as pltpu
from jax.experimental.pallas import tpu_sc as plsc

_NC = 2
_NS = 16
_NW = _NC * _NS

_K = 80


def _sc_degree(n_nodes_pad, n_chunks):
    mesh = plsc.VectorSubcoreMesh(
        core_axis_name="c", subcore_axis_name="s",
        num_cores=_NC, num_subcores=_NS)
    rows_per_tile = n_nodes_pad // _NS

    @functools.partial(
        pl.kernel,
        out_type=jax.ShapeDtypeStruct((_NC * n_nodes_pad,), jnp.float32),
        mesh=mesh,
        scratch_types=[
            pltpu.VMEM_SHARED((n_nodes_pad,), jnp.float32),
            pltpu.VMEM((n_chunks, _K), jnp.int32),
            pltpu.VMEM((_K,), jnp.float32),
            pltpu.SemaphoreType.DMA,
            pltpu.SemaphoreType.DMA,
        ],
    )
    def deg_kernel(dst_hbm, ones_hbm, out_hbm, acc_sh, idx_v, ones_v,
                   ssem0, ssem1):
        cid = lax.axis_index("c")
        sid = lax.axis_index("s")
        wid = cid * _NS + sid
        r0 = pl.multiple_of(sid * rows_per_tile, 128)
        pltpu.sync_copy(ones_hbm.at[pl.ds(r0, rows_per_tile)],
                        acc_sh.at[pl.ds(r0, rows_per_tile)])
        pltpu.sync_copy(dst_hbm.at[wid], idx_v)
        pltpu.sync_copy(ones_hbm.at[pl.ds(0, _K)], ones_v)
        plsc.subcore_barrier()

        ssems = (ssem0, ssem1)

        def swait(j, s):
            pltpu.make_async_copy(ones_v, acc_sh.at[idx_v.at[j]],
                                  ssems[s]).wait()

        pltpu.async_copy(ones_v, acc_sh.at[idx_v.at[0]], ssems[0], add=True)

        def body(i, carry):
            for p in range(2):
                j = 2 * i + 1 + p
                swait(j - 1, p)
                pltpu.async_copy(ones_v, acc_sh.at[idx_v.at[j]],
                                 ssems[(1 + p) % 2], add=True)
            return carry

        n2 = (n_chunks - 1) // 2
        lax.fori_loop(0, n2, body, 0)
        for j in range(2 * n2 + 1, n_chunks):
            swait(j - 1, (j - 1) % 2)
            pltpu.async_copy(ones_v, acc_sh.at[idx_v.at[j]],
                             ssems[j % 2], add=True)
        swait(n_chunks - 1, (n_chunks - 1) % 2)
        plsc.subcore_barrier()
        o0 = pl.multiple_of(cid * n_nodes_pad + r0, 128)
        pltpu.sync_copy(acc_sh.at[pl.ds(r0, rows_per_tile)],
                        out_hbm.at[pl.ds(o0, rows_per_tile)])

    return deg_kernel


def _sc_edge_pass(n_nodes, d_feat, n_chunks):
    mesh = plsc.VectorSubcoreMesh(
        core_axis_name="c", subcore_axis_name="s",
        num_cores=_NC, num_subcores=_NS)
    rows_main = (n_nodes // (8 * _NS)) * 8
    rows_last = n_nodes - rows_main * (_NS - 1)
    assert rows_last % 8 == 0

    @functools.partial(
        pl.kernel,
        out_type=jax.ShapeDtypeStruct((_NC, n_nodes, d_feat), jnp.float32),
        mesh=mesh,
        scratch_types=[
            pltpu.VMEM_SHARED((n_nodes + 8, d_feat), jnp.float32),
            [pltpu.VMEM((2, _K), jnp.int32)] * 4,
            [pltpu.VMEM((_K, d_feat), jnp.float32)] * 4,
            [pltpu.SemaphoreType.DMA] * 4,
            [pltpu.SemaphoreType.DMA] * 4,
            [pltpu.SemaphoreType.DMA] * 4,
        ],
    )
    def edge_kernel(g_hbm, zeros_hbm, ei_hbm, out_hbm, acc_sh, idxs, bufs,
                    isems, gsems, ssems):
        cid = lax.axis_index("c")
        sid = lax.axis_index("s")
        wid = cid * _NS + sid
        r0 = pl.multiple_of(sid * rows_main, 8)

        for c, init_hbm in ((0, g_hbm), (1, zeros_hbm)):
            @pl.when(jnp.logical_and(cid == c, sid < _NS - 1))
            def _():
                pltpu.sync_copy(init_hbm.at[pl.ds(r0, rows_main)],
                                acc_sh.at[pl.ds(r0, rows_main)])

            @pl.when(jnp.logical_and(cid == c, sid == _NS - 1))
            def _():
                pltpu.sync_copy(
                    init_hbm.at[pl.ds(rows_main * (_NS - 1), rows_last)],
                    acc_sh.at[pl.ds(rows_main * (_NS - 1), rows_last)])

        plsc.subcore_barrier()

        def wait_idx(j, s):
            pltpu.make_async_copy(ei_hbm.at[wid, j], idxs[s], isems[s]).wait()

        def wait_gather(s):
            pltpu.make_async_copy(g_hbm.at[idxs[s].at[0]], bufs[s],
                                  gsems[s]).wait()

        def wait_scatter(s):
            pltpu.make_async_copy(bufs[s], acc_sh.at[idxs[s].at[1]],
                                  ssems[s]).wait()

        pltpu.async_copy(ei_hbm.at[wid, 0], idxs[0], isems[0])
        if n_chunks > 1:
            pltpu.async_copy(ei_hbm.at[wid, 1], idxs[1], isems[1])
        wait_idx(0, 0)
        pltpu.async_copy(g_hbm.at[idxs[0].at[0]], bufs[0], gsems[0])

        def step(j, p, static):
            if static:
                if j + 1 < n_chunks:
                    wait_idx(j + 1, (p + 1) % 4)
                    pltpu.async_copy(g_hbm.at[idxs[(p + 1) % 4].at[0]],
                                     bufs[(p + 1) % 4], gsems[(p + 1) % 4])
                wait_gather(p)
                if j >= 1:
                    wait_scatter((p + 3) % 4)
                pltpu.async_copy(bufs[p], acc_sh.at[idxs[p].at[1]], ssems[p],
                                 add=True)
                if j + 2 < n_chunks:
                    pltpu.async_copy(ei_hbm.at[wid, j + 2], idxs[(p + 2) % 4],
                                     isems[(p + 2) % 4])
            else:
                wait_idx(j + 1, (p + 1) % 4)
                pltpu.async_copy(g_hbm.at[idxs[(p + 1) % 4].at[0]],
                                 bufs[(p + 1) % 4], gsems[(p + 1) % 4])
                wait_gather(p)

                @pl.when(j >= 1)
                def _():
                    wait_scatter((p + 3) % 4)

                pltpu.async_copy(bufs[p], acc_sh.at[idxs[p].at[1]], ssems[p],
                                 add=True)

                @pl.when(j + 2 < n_chunks)
                def _():
                    pltpu.async_copy(ei_hbm.at[wid, j + 2], idxs[(p + 2) % 4],
                                     isems[(p + 2) % 4])

        n4 = (n_chunks - 1) // 4

        def body4(i, carry):
            for p in range(4):
                step(4 * i + p, p, False)
            return carry

        lax.fori_loop(0, n4, body4, 0)
        for j in range(4 * n4, n_chunks):
            step(j, j % 4, True)
        wait_scatter((n_chunks - 1) % 4)
        plsc.subcore_barrier()

        @pl.when(sid < _NS - 1)
        def _():
            pltpu.sync_copy(acc_sh.at[pl.ds(r0, rows_main)],
                            out_hbm.at[cid, pl.ds(r0, rows_main)])

        @pl.when(sid == _NS - 1)
        def _():
            pltpu.sync_copy(
                acc_sh.at[pl.ds(rows_main * (_NS - 1), rows_last)],
                out_hbm.at[cid, pl.ds(rows_main * (_NS - 1), rows_last)])

    return edge_kernel


def _tc_first(x, w1, d0, d1, block_rows):
    n, d = x.shape
    grid = n // block_rows

    def kern(x_ref, w_ref, d0_ref, d1_ref, g_ref, dinv_ref):
        deg = d0_ref[...] + d1_ref[...] - 1.0
        dinv = lax.rsqrt(deg)
        g_ref[...] = jnp.dot(x_ref[...], w_ref[...],
                             preferred_element_type=jnp.float32) * dinv
        dinv_ref[...] = dinv

    return pl.pallas_call(
        kern,
        grid=(grid,),
        in_specs=[
            pl.BlockSpec((block_rows, d), lambda i: (i, 0)),
            pl.BlockSpec((d, d), lambda i: (0, 0)),
            pl.BlockSpec((block_rows, 1), lambda i: (i, 0)),
            pl.BlockSpec((block_rows, 1), lambda i: (i, 0)),
        ],
        out_specs=[
            pl.BlockSpec((block_rows, d), lambda i: (i, 0)),
            pl.BlockSpec((block_rows, 1), lambda i: (i, 0)),
        ],
        out_shape=[
            jax.ShapeDtypeStruct((n, d), jnp.float32),
            jax.ShapeDtypeStruct((n, 1), jnp.float32),
        ],
    )(x, w1, d0, d1)


def _tc_mid(a0, a1, dinv, b, w, block_rows):
    n, d = a0.shape
    grid = n // block_rows

    def kern(a0_ref, a1_ref, dinv_ref, b_ref, w_ref, g_ref):
        dinv = dinv_ref[...]
        acc = a0_ref[...] + a1_ref[...]
        h = jnp.maximum(acc * dinv + b_ref[...], 0.0)
        g_ref[...] = jnp.dot(h, w_ref[...],
                             preferred_element_type=jnp.float32) * dinv

    return pl.pallas_call(
        kern,
        grid=(grid,),
        in_specs=[
            pl.BlockSpec((block_rows, d), lambda i: (i, 0)),
            pl.BlockSpec((block_rows, d), lambda i: (i, 0)),
            pl.BlockSpec((block_rows, 1), lambda i: (i, 0)),
            pl.BlockSpec((1, d), lambda i: (0, 0)),
            pl.BlockSpec((d, d), lambda i: (0, 0)),
        ],
        out_specs=pl.BlockSpec((block_rows, d), lambda i: (i, 0)),
        out_shape=jax.ShapeDtypeStruct((n, d), jnp.float32),
    )(a0, a1, dinv, b, w)


def _tc_final(a0, a1, dinv, b, wl, bl, batch3d, n_graphs, block_rows):
    n, d = a0.shape
    grid = n // block_rows

    def kern(a0_ref, a1_ref, dinv_ref, b_ref, wl_ref, bl_ref,
             batch_ref, out_ref):
        i = pl.program_id(0)
        acc = a0_ref[...] + a1_ref[...]
        h = jnp.maximum(acc * dinv_ref[...] + b_ref[...], 0.0)
        s = jnp.dot(h, wl_ref[...], preferred_element_type=jnp.float32)
        bidx = batch_ref[0, 0, :]
        gids = lax.broadcasted_iota(jnp.int32, (block_rows, n_graphs), 1)
        onehot = (bidx[:, None] == gids).astype(jnp.float32)
        partial = lax.dot_general(
            onehot, s, (((0,), (0,)), ((), ())),
            preferred_element_type=jnp.float32)

        @pl.when(i == 0)
        def _():
            out_ref[...] = jnp.broadcast_to(bl_ref[...], (n_graphs, 1))

        out_ref[...] += partial

    return pl.pallas_call(
        kern,
        grid=(grid,),
        in_specs=[
            pl.BlockSpec((block_rows, d), lambda i: (i, 0)),
            pl.BlockSpec((block_rows, d), lambda i: (i, 0)),
            pl.BlockSpec((block_rows, 1), lambda i: (i, 0)),
            pl.BlockSpec((1, d), lambda i: (0, 0)),
            pl.BlockSpec((d, 1), lambda i: (0, 0)),
            pl.BlockSpec((1, 1), lambda i: (0, 0)),
            pl.BlockSpec((1, 1, block_rows), lambda i: (i, 0, 0)),
        ],
        out_specs=pl.BlockSpec((n_graphs, 1), lambda i: (0, 0)),
        out_shape=jax.ShapeDtypeStruct((n_graphs, 1), jnp.float32),
    )(a0, a1, dinv, b, wl, bl, batch3d)


def kernel(x, edge_index, batch, W1, b1, W2, b2, W3, b3, Wl, bl):
    n, d = x.shape
    n_edges = edge_index.shape[1]
    n_graphs = 64

    epw = n_edges // _NW
    assert epw * _NW == n_edges
    n_chunks = -(-epw // _K)
    pad = n_chunks * _K - epw

    n_pad = ((n + _NS * 128 - 1) // (_NS * 128)) * (_NS * 128)

    src2 = edge_index[0].reshape(_NW, epw)
    dst2 = edge_index[1].reshape(_NW, epw)
    if pad:
        src2 = jnp.pad(src2, ((0, 0), (0, pad)))
        dst2 = jnp.pad(dst2, ((0, 0), (0, pad)), constant_values=n)
    ei = jnp.concatenate([src2.reshape(_NW, n_chunks, 1, _K),
                         dst2.reshape(_NW, n_chunks, 1, _K)], axis=2)
    dst3 = dst2.reshape(_NW, n_chunks, _K)
    ones = jnp.ones((n_pad,), jnp.float32)

    degs = _sc_degree(n_pad, n_chunks)(dst3, ones)
    d0 = degs[:n].reshape(n, 1)
    d1 = degs[n_pad:n_pad + n].reshape(n, 1)

    block_rows = 2000
    b1r = b1.reshape(1, d)
    b2r = b2.reshape(1, d)
    b3r = b3.reshape(1, d)
    blr = bl.reshape(1, 1)
    batch3d = batch.reshape(n // block_rows, 1, block_rows)

    edge_pass = _sc_edge_pass(n, d, n_chunks)
    zeros = jnp.zeros((n, d), jnp.float32)

    g1, dinv = _tc_first(x, W1, d0, d1, block_rows)
    a = edge_pass(g1, zeros, ei)
    g2 = _tc_mid(a[0], a[1], dinv, b1r, W2, block_rows)
    a = edge_pass(g2, zeros, ei)
    g3 = _tc_mid(a[0], a[1], dinv, b2r, W3, block_rows)
    a = edge_pass(g3, zeros, ei)
    out = _tc_final(a[0], a[1], dinv, b3r, Wl, blr, batch3d,
                    n_graphs, block_rows)
    return out
